# baseline (speedup 1.0000x reference)
import jax
import jax.numpy as jnp
from jax.experimental import pallas as pl

_N = 10000
_H = 128
_M = 64
_N_LAYER_ARGS = 19


def _fused_body(*refs):
    f32 = jnp.float32
    x_ref, win_ref, bin_ref, mem_ref = refs[0:4]
    l0 = refs[4:4 + _N_LAYER_ARGS]
    l1 = refs[4 + _N_LAYER_ARGS:4 + 2 * _N_LAYER_ARGS]
    wc1_ref, bc1_ref, wc2_ref, bc2_ref = refs[4 + 2 * _N_LAYER_ARGS:-1]
    out_ref = refs[-1]

    h = jnp.dot(x_ref[...], win_ref[...], preferred_element_type=f32)
    h = h + bin_ref[...]
    mem = mem_ref[...]
    for (wih_r, wih_z, wih_n, whh_r, whh_z, whh_n,
         bi_r, bi_z, bi_n, bh_r, bh_z, bh_n,
         ah, am, bm1, wm2, bm2, wa, ba) in (l0, l1):
        xm = jnp.sum(h, axis=0, keepdims=True) * (1.0 / _N)
        gi_r = jnp.dot(xm, wih_r[...], preferred_element_type=f32) + bi_r[...]
        gi_z = jnp.dot(xm, wih_z[...], preferred_element_type=f32) + bi_z[...]
        gi_n = jnp.dot(xm, wih_n[...], preferred_element_type=f32) + bi_n[...]
        gh_r = jnp.dot(mem, whh_r[...], preferred_element_type=f32) + bh_r[...]
        gh_z = jnp.dot(mem, whh_z[...], preferred_element_type=f32) + bh_z[...]
        gh_n = jnp.dot(mem, whh_n[...], preferred_element_type=f32) + bh_n[...]
        r = jax.nn.sigmoid(gi_r + gh_r)
        z = jax.nn.sigmoid(gi_z + gh_z)
        n = jnp.tanh(gi_n + r * gh_n)
        mem = (1.0 - z) * n + z * mem
        c = jnp.dot(mem, am[...], preferred_element_type=f32) + bm1[...]
        u = jnp.maximum(
            jnp.dot(h, ah[...], preferred_element_type=f32) + c, 0.0)
        msg = jnp.dot(u, wm2[...], preferred_element_type=f32) + bm2[...]
        h = jnp.dot(msg, wa[...], preferred_element_type=f32) + ba[...]
    v = jnp.maximum(
        jnp.dot(h, wc1_ref[...], preferred_element_type=f32) + bc1_ref[...],
        0.0)
    out_ref[...] = (jnp.dot(v, wc2_ref[...], preferred_element_type=f32)
                    + bc2_ref[...])


def kernel(x, edge_index, edge_attr, t, W_in, b_in, memory,
           l0_wih, l0_whh, l0_bih, l0_bhh, l0_Wm1, l0_bm1, l0_Wm2, l0_bm2,
           l0_Wa, l0_ba,
           l1_wih, l1_whh, l1_bih, l1_bhh, l1_Wm1, l1_bm1, l1_Wm2, l1_bm2,
           l1_Wa, l1_ba,
           Wc1, bc1, Wc2, bc2):
    del edge_index, edge_attr, t
    f32 = jnp.float32

    def row(v):
        return v.reshape(1, -1).astype(f32)

    args = [x.astype(f32), W_in.T.astype(f32), row(b_in), memory.astype(f32)]
    for (wih, whh, bih, bhh, Wm1, bm1, Wm2, bm2, Wa, ba) in (
        (l0_wih, l0_whh, l0_bih, l0_bhh, l0_Wm1, l0_bm1, l0_Wm2, l0_bm2,
         l0_Wa, l0_ba),
        (l1_wih, l1_whh, l1_bih, l1_bhh, l1_Wm1, l1_bm1, l1_Wm2, l1_bm2,
         l1_Wa, l1_ba)):
        wih_t = wih.T.astype(f32)
        whh_t = whh.T.astype(f32)
        args += [wih_t[:, 0:_M], wih_t[:, _M:2 * _M], wih_t[:, 2 * _M:],
                 whh_t[:, 0:_M], whh_t[:, _M:2 * _M], whh_t[:, 2 * _M:],
                 row(bih[0:_M]), row(bih[_M:2 * _M]), row(bih[2 * _M:]),
                 row(bhh[0:_M]), row(bhh[_M:2 * _M]), row(bhh[2 * _M:]),
                 Wm1[:, :_H].T.astype(f32),
                 Wm1[:, _H:].T.astype(f32),
                 row(bm1),
                 Wm2.T.astype(f32), row(bm2),
                 Wa.T.astype(f32), row(ba)]
    args += [Wc1.T.astype(f32), row(bc1), Wc2.T.astype(f32), row(bc2)]

    return pl.pallas_call(
        _fused_body,
        out_shape=jax.ShapeDtypeStruct((_N, 2), f32),
    )(*args)


# speedup vs baseline: 1.6682x; 1.6682x over previous
import jax
import jax.numpy as jnp
from jax import lax
from jax.experimental import pallas as pl

_N = 10000
_H = 128
_M = 64

_DN_T = (((1,), (1,)), ((), ()))


def _dot_t(a, b):
    return lax.dot_general(a, b, _DN_T, preferred_element_type=jnp.float32)


def _fused_body(x_ref, win_ref, bin_ref, mem_ref,
                l0_wih, l0_whh, l0_bih, l0_bhh, l0_wm1, l0_bm1, l0_wm2,
                l0_bm2, l0_wa, l0_ba,
                l1_wih, l1_whh, l1_bih, l1_bhh, l1_wm1, l1_bm1, l1_wm2,
                l1_bm2, l1_wa, l1_ba,
                wc1_ref, bc1_ref, wc2_ref, bc2_ref, out_ref):
    h = _dot_t(x_ref[...], win_ref[...]) + bin_ref[...]
    mem = mem_ref[...]
    for (wih, whh, bih, bhh, wm1, bm1, wm2, bm2, wa, ba) in (
            (l0_wih, l0_whh, l0_bih, l0_bhh, l0_wm1, l0_bm1, l0_wm2,
             l0_bm2, l0_wa, l0_ba),
            (l1_wih, l1_whh, l1_bih, l1_bhh, l1_wm1, l1_bm1, l1_wm2,
             l1_bm2, l1_wa, l1_ba)):
        xm = jnp.sum(h, axis=0, keepdims=True) * (1.0 / _N)
        gi_r = _dot_t(xm, wih[0:_M, :]) + bih[:, 0:_M]
        gi_z = _dot_t(xm, wih[_M:2 * _M, :]) + bih[:, _M:2 * _M]
        gi_n = _dot_t(xm, wih[2 * _M:, :]) + bih[:, 2 * _M:]
        gh_r = _dot_t(mem, whh[0:_M, :]) + bhh[:, 0:_M]
        gh_z = _dot_t(mem, whh[_M:2 * _M, :]) + bhh[:, _M:2 * _M]
        gh_n = _dot_t(mem, whh[2 * _M:, :]) + bhh[:, 2 * _M:]
        r = jax.nn.sigmoid(gi_r + gh_r)
        z = jax.nn.sigmoid(gi_z + gh_z)
        n = jnp.tanh(gi_n + r * gh_n)
        mem = (1.0 - z) * n + z * mem
        c = _dot_t(mem, wm1[:, _H:]) + bm1[...]
        u = jnp.maximum(_dot_t(h, wm1[:, 0:_H]) + c, 0.0)
        msg = _dot_t(u, wm2[...]) + bm2[...]
        h = _dot_t(msg, wa[...]) + ba[...]
    v = jnp.maximum(_dot_t(h, wc1_ref[...]) + bc1_ref[...], 0.0)
    out_ref[...] = _dot_t(v, wc2_ref[...]) + bc2_ref[...]


def kernel(x, edge_index, edge_attr, t, W_in, b_in, memory,
           l0_wih, l0_whh, l0_bih, l0_bhh, l0_Wm1, l0_bm1, l0_Wm2, l0_bm2,
           l0_Wa, l0_ba,
           l1_wih, l1_whh, l1_bih, l1_bhh, l1_Wm1, l1_bm1, l1_Wm2, l1_bm2,
           l1_Wa, l1_ba,
           Wc1, bc1, Wc2, bc2):
    del edge_index, edge_attr, t

    def row(v):
        return v.reshape(1, -1)

    return pl.pallas_call(
        _fused_body,
        out_shape=jax.ShapeDtypeStruct((_N, 2), jnp.float32),
    )(x, W_in, row(b_in), memory,
      l0_wih, l0_whh, row(l0_bih), row(l0_bhh), l0_Wm1, row(l0_bm1),
      l0_Wm2, row(l0_bm2), l0_Wa, row(l0_ba),
      l1_wih, l1_whh, row(l1_bih), row(l1_bhh), l1_Wm1, row(l1_bm1),
      l1_Wm2, row(l1_bm2), l1_Wa, row(l1_ba),
      Wc1, row(bc1), Wc2, row(bc2))
